# Initial kernel scaffold; baseline (speedup 1.0000x reference)
import functools

import jax
import jax.numpy as jnp
from jax import lax
from jax.experimental import pallas as pl
from jax.experimental.pallas import tpu as pltpu
from jax.experimental.pallas import tpu_sc as plsc

B, S, H = 4096, 200, 64
N = B * S
VOCAB = 100000
AGE_VOCAB = 120
TYPE_VOCAB = 2
EPS = 1e-12

NC, NS = 2, 16
NW = NC * NS
PER_W = N // NW
CHUNK = 512
KROWS = CHUNK // 128
NCHUNKS = PER_W // CHUNK
GRP = 16
NGRP = CHUNK // GRP


def _rsqrt(x):
    i = plsc.bitcast(x, jnp.int32)
    i = jnp.int32(0x5F3759DF) - (i >> 1)
    y = plsc.bitcast(i, jnp.float32)
    for _ in range(3):
        y = y * (1.5 - 0.5 * x * y * y)
    return y


def _body(concept_hbm, cidx_hbm, aidx_hbm, tidx_hbm, age_hbm, seg_hbm,
          gam_hbm, bet_hbm, out_hbm,
          age_v, seg_v, gam_v, bet_v, cidx_v, aidx_v, tidx_v, rows_v, sem):
    wid = lax.axis_index("s") * NC + lax.axis_index("c")

    pltpu.sync_copy(age_hbm, age_v)
    pltpu.sync_copy(seg_hbm, seg_v)
    pltpu.sync_copy(gam_hbm, gam_v)
    pltpu.sync_copy(bet_hbm, bet_v)

    lanes = lax.iota(jnp.int32, GRP)

    def chunk_body(c, _):
        base = wid * PER_W + c * CHUNK
        krow = base // 128

        pltpu.sync_copy(cidx_hbm.at[pl.ds(krow, KROWS)], cidx_v)
        pltpu.sync_copy(aidx_hbm.at[pl.ds(krow, KROWS)], aidx_v)
        pltpu.sync_copy(tidx_hbm.at[pl.ds(krow, KROWS)], tidx_v)

        cps = [pltpu.async_copy(concept_hbm.at[cidx_v.at[k]],
                                rows_v.at[pl.ds(k * 128, 128)], sem)
               for k in range(KROWS)]
        for cp in cps:
            cp.wait()

        def group_body(g, _):
            r0 = g * GRP
            rvec = r0 + lanes
            avec = aidx_v[r0 // 128, pl.ds(r0 % 128, GRP)]
            tvec = tidx_v[r0 // 128, pl.ds(r0 % 128, GRP)]

            s = jnp.zeros((GRP,), jnp.float32)
            s2 = jnp.zeros((GRP,), jnp.float32)
            for j in range(H):
                jf = jnp.full((GRP,), j, jnp.int32)
                v = plsc.load_gather(rows_v, [rvec, jf])
                v = v + plsc.load_gather(age_v, [avec, jf])
                v = v + plsc.load_gather(seg_v, [tvec, jf])
                plsc.store_scatter(rows_v, [rvec, jf], v)
                s = s + v
                s2 = s2 + v * v

            mean = s * (1.0 / H)
            var = s2 * (1.0 / H) - mean * mean
            rstd = _rsqrt(var + EPS)

            for j in range(H):
                jf = jnp.full((GRP,), j, jnp.int32)
                v = plsc.load_gather(rows_v, [rvec, jf])
                gj = plsc.load_gather(gam_v, [jf])
                bj = plsc.load_gather(bet_v, [jf])
                o = (v - mean) * rstd * gj + bj
                plsc.store_scatter(rows_v, [rvec, jf], o)
            return 0

        lax.fori_loop(0, NGRP, group_body, 0)
        pltpu.sync_copy(rows_v, out_hbm.at[pl.ds(base, CHUNK)])
        return 0

    lax.fori_loop(0, NCHUNKS, chunk_body, 0)


_mesh = plsc.VectorSubcoreMesh(core_axis_name="c", subcore_axis_name="s")

_sc_call = functools.partial(
    pl.kernel,
    out_type=jax.ShapeDtypeStruct((N, H), jnp.float32),
    mesh=_mesh,
    scratch_types=[
        pltpu.VMEM((AGE_VOCAB, H), jnp.float32),
        pltpu.VMEM((TYPE_VOCAB, H), jnp.float32),
        pltpu.VMEM((H,), jnp.float32),
        pltpu.VMEM((H,), jnp.float32),
        pltpu.VMEM((KROWS, 128), jnp.int32),
        pltpu.VMEM((KROWS, 128), jnp.int32),
        pltpu.VMEM((KROWS, 128), jnp.int32),
        pltpu.VMEM((CHUNK, H), jnp.float32),
        pltpu.SemaphoreType.DMA,
    ],
)(_body)


@jax.jit
def kernel(input_ids, token_type_ids, position_ids, concept_table,
           age_table, segment_table, ln_gamma, ln_beta):
    cidx = input_ids.astype(jnp.int32).reshape(N // 128, 128)
    aidx = position_ids.astype(jnp.int32).reshape(N // 128, 128)
    tidx = token_type_ids.astype(jnp.int32).reshape(N // 128, 128)
    out = _sc_call(concept_table, cidx, aidx, tidx, age_table,
                   segment_table, ln_gamma, ln_beta)
    return out.reshape(B, S, H)


# speedup vs baseline: 1.2703x; 1.2703x over previous
import functools

import jax
import jax.numpy as jnp
from jax import lax
from jax.experimental import pallas as pl
from jax.experimental.pallas import tpu as pltpu
from jax.experimental.pallas import tpu_sc as plsc

B, S, H = 4096, 200, 64
N = B * S
VOCAB = 100000
AGE_VOCAB = 120
TYPE_VOCAB = 2
EPS = 1e-12

NC, NS = 2, 16
NW = NC * NS
PER_W = N // NW
CHUNK = 1024
KROWS = CHUNK // 128
NCHUNKS = PER_W // CHUNK
GRP = 16
NGRP = CHUNK // GRP


def _rsqrt(x):
    i = plsc.bitcast(x, jnp.int32)
    i = jnp.int32(0x5F3759DF) - (i >> 1)
    y = plsc.bitcast(i, jnp.float32)
    for _ in range(3):
        y = y * (1.5 - 0.5 * x * y * y)
    return y


def _body(concept_hbm, cidx_hbm, aidx_hbm, tidx_hbm, age_hbm, seg_hbm,
          gam_hbm, bet_hbm, out_hbm,
          age_v, seg_v, gam_v, bet_v, cidx_v, aidx_v, tidx_v, rows_v, sem):
    wid = lax.axis_index("s") * NC + lax.axis_index("c")

    pltpu.sync_copy(age_hbm, age_v)
    pltpu.sync_copy(seg_hbm, seg_v)
    pltpu.sync_copy(gam_hbm, gam_v)
    pltpu.sync_copy(bet_hbm, bet_v)

    lanes = lax.iota(jnp.int32, GRP)

    def chunk_body(c, _):
        base = pl.multiple_of(wid * PER_W + c * CHUNK, 8)
        krow = pl.multiple_of(base // 128, 8)

        pltpu.sync_copy(cidx_hbm.at[pl.ds(krow, KROWS)], cidx_v)
        pltpu.sync_copy(aidx_hbm.at[pl.ds(krow, KROWS)], aidx_v)
        pltpu.sync_copy(tidx_hbm.at[pl.ds(krow, KROWS)], tidx_v)

        cps = [pltpu.async_copy(concept_hbm.at[cidx_v.at[k]],
                                rows_v.at[pl.ds(k * 128, 128)], sem)
               for k in range(KROWS)]
        for cp in cps:
            cp.wait()

        def group_body(g, _):
            r0 = g * GRP
            rvec = r0 + lanes
            avec = aidx_v[r0 // 128, pl.ds(r0 % 128, GRP)]
            tvec = tidx_v[r0 // 128, pl.ds(r0 % 128, GRP)]

            s = jnp.zeros((GRP,), jnp.float32)
            s2 = jnp.zeros((GRP,), jnp.float32)
            for j in range(H):
                jf = jnp.full((GRP,), j, jnp.int32)
                v = plsc.load_gather(rows_v, [rvec, jf])
                v = v + plsc.load_gather(age_v, [avec, jf])
                v = v + plsc.load_gather(seg_v, [tvec, jf])
                plsc.store_scatter(rows_v, [rvec, jf], v)
                s = s + v
                s2 = s2 + v * v

            mean = s * (1.0 / H)
            var = s2 * (1.0 / H) - mean * mean
            rstd = _rsqrt(var + EPS)

            for j in range(H):
                jf = jnp.full((GRP,), j, jnp.int32)
                v = plsc.load_gather(rows_v, [rvec, jf])
                gj = plsc.load_gather(gam_v, [jf])
                bj = plsc.load_gather(bet_v, [jf])
                o = (v - mean) * rstd * gj + bj
                plsc.store_scatter(rows_v, [rvec, jf], o)
            return 0

        lax.fori_loop(0, NGRP, group_body, 0)
        pltpu.sync_copy(rows_v, out_hbm.at[pl.ds(base, CHUNK)])
        return 0

    lax.fori_loop(0, NCHUNKS, chunk_body, 0)


_mesh = plsc.VectorSubcoreMesh(core_axis_name="c", subcore_axis_name="s")

_sc_call = functools.partial(
    pl.kernel,
    out_type=jax.ShapeDtypeStruct((N, H), jnp.float32),
    mesh=_mesh,
    compiler_params=pltpu.CompilerParams(
        needs_layout_passes=False, use_tc_tiling_on_sc=False),
    scratch_types=[
        pltpu.VMEM((AGE_VOCAB, H), jnp.float32),
        pltpu.VMEM((TYPE_VOCAB, H), jnp.float32),
        pltpu.VMEM((H,), jnp.float32),
        pltpu.VMEM((H,), jnp.float32),
        pltpu.VMEM((KROWS, 128), jnp.int32),
        pltpu.VMEM((KROWS, 128), jnp.int32),
        pltpu.VMEM((KROWS, 128), jnp.int32),
        pltpu.VMEM((CHUNK, H), jnp.float32),
        pltpu.SemaphoreType.DMA,
    ],
)(_body)


@jax.jit
def kernel(input_ids, token_type_ids, position_ids, concept_table,
           age_table, segment_table, ln_gamma, ln_beta):
    cidx = input_ids.astype(jnp.int32).reshape(N // 128, 128)
    aidx = position_ids.astype(jnp.int32).reshape(N // 128, 128)
    tidx = token_type_ids.astype(jnp.int32).reshape(N // 128, 128)
    out = _sc_call(concept_table, cidx, aidx, tidx, age_table,
                   segment_table, ln_gamma, ln_beta)
    return out.reshape(B, S, H)


# speedup vs baseline: 5.4840x; 4.3170x over previous
import functools

import jax
import jax.numpy as jnp
from jax import lax
from jax.experimental import pallas as pl
from jax.experimental.pallas import tpu as pltpu
from jax.experimental.pallas import tpu_sc as plsc

B, S, H = 4096, 200, 64
N = B * S
VOCAB = 100000
AGE_VOCAB = 120
TYPE_VOCAB = 2
EPS = 1e-12

NC, NS = 2, 16
NW = NC * NS
PER_W = N // NW
PAIR = 1024
HALF = 512
NPAIR = PER_W // PAIR
KSUB = 4


def _gather_body(concept_hbm, cidx_hbm, out_hbm, idx_v, buf0, buf1,
                 sg0, sg1, sw0, sw1):
    wid = lax.axis_index("s") * NC + lax.axis_index("c")

    def pair_body(p, _):
        base = pl.multiple_of(wid * PER_W + p * PAIR, 8)
        krow = pl.multiple_of(base // 128, 8)
        pltpu.sync_copy(cidx_hbm.at[pl.ds(krow, 8)], idx_v)

        g0 = [pltpu.async_copy(concept_hbm.at[idx_v.at[k]],
                               buf0.at[pl.ds(k * 128, 128)], sg0)
              for k in range(KSUB)]
        g1 = [pltpu.async_copy(concept_hbm.at[idx_v.at[KSUB + k]],
                               buf1.at[pl.ds(k * 128, 128)], sg1)
              for k in range(KSUB)]
        for cp in g0:
            cp.wait()
        w0 = pltpu.async_copy(buf0, out_hbm.at[pl.ds(base, HALF)], sw0)
        for cp in g1:
            cp.wait()
        w1 = pltpu.async_copy(buf1, out_hbm.at[pl.ds(base + HALF, HALF)], sw1)
        w0.wait()
        w1.wait()
        return 0

    lax.fori_loop(0, NPAIR, pair_body, 0)


_sc_gather = functools.partial(
    pl.kernel,
    out_type=jax.ShapeDtypeStruct((N, H), jnp.float32),
    mesh=plsc.VectorSubcoreMesh(core_axis_name="c", subcore_axis_name="s"),
    compiler_params=pltpu.CompilerParams(
        needs_layout_passes=False, use_tc_tiling_on_sc=False),
    scratch_types=[
        pltpu.VMEM((8, 128), jnp.int32),
        pltpu.VMEM((HALF, H), jnp.float32),
        pltpu.VMEM((HALF, H), jnp.float32),
        pltpu.SemaphoreType.DMA,
        pltpu.SemaphoreType.DMA,
        pltpu.SemaphoreType.DMA,
        pltpu.SemaphoreType.DMA,
    ],
)(_gather_body)


TB = 1024
NTB = N // TB
IDR = TB // 128


def _ln_body(g_ref, pos_ref, tt_ref, age_ref, seg_ref, gam_ref, bet_ref,
             o_ref):
    pos3 = pos_ref[...][:, :, None]
    tt3 = tt_ref[...][:, :, None]

    iota3 = lax.broadcasted_iota(jnp.int32, (IDR, 128, AGE_VOCAB), 2)
    onehot = (pos3 == iota3).astype(jnp.float32).reshape(TB, AGE_VOCAB)
    age_part = lax.dot_general(
        onehot, age_ref[...],
        (((1,), (0,)), ((), ())),
        preferred_element_type=jnp.float32,
        precision=lax.Precision.HIGHEST)

    s = seg_ref[...]
    seg_part = jnp.where(tt3 == 0, s[0:1, :][None], s[1:2, :][None])
    seg_part = seg_part.reshape(TB, H)

    x = g_ref[...] + age_part + seg_part
    mean = jnp.mean(x, axis=-1, keepdims=True)
    cx = x - mean
    var = jnp.mean(cx * cx, axis=-1, keepdims=True)
    y = cx * lax.rsqrt(var + EPS)
    o_ref[...] = y * gam_ref[...] + bet_ref[...]


_tc_ln = pl.pallas_call(
    _ln_body,
    grid=(NTB,),
    in_specs=[
        pl.BlockSpec((TB, H), lambda i: (i, 0)),
        pl.BlockSpec((IDR, 128), lambda i: (i, 0)),
        pl.BlockSpec((IDR, 128), lambda i: (i, 0)),
        pl.BlockSpec((AGE_VOCAB, H), lambda i: (0, 0)),
        pl.BlockSpec((TYPE_VOCAB, H), lambda i: (0, 0)),
        pl.BlockSpec((1, H), lambda i: (0, 0)),
        pl.BlockSpec((1, H), lambda i: (0, 0)),
    ],
    out_specs=pl.BlockSpec((TB, H), lambda i: (i, 0)),
    out_shape=jax.ShapeDtypeStruct((N, H), jnp.float32),
    compiler_params=pltpu.CompilerParams(
        dimension_semantics=("arbitrary",)),
)


@jax.jit
def kernel(input_ids, token_type_ids, position_ids, concept_table,
           age_table, segment_table, ln_gamma, ln_beta):
    cidx = input_ids.astype(jnp.int32).reshape(N // 128, 128)
    pos = position_ids.astype(jnp.int32).reshape(N // 128, 128)
    tt = token_type_ids.astype(jnp.int32).reshape(N // 128, 128)
    gathered = _sc_gather(concept_table, cidx)
    out = _tc_ln(gathered, pos, tt, age_table, segment_table,
                 ln_gamma.reshape(1, H), ln_beta.reshape(1, H))
    return out.reshape(B, S, H)


# speedup vs baseline: 6.7860x; 1.2374x over previous
import functools

import jax
import jax.numpy as jnp
from jax import lax
from jax.experimental import pallas as pl
from jax.experimental.pallas import tpu as pltpu
from jax.experimental.pallas import tpu_sc as plsc

B, S, H = 4096, 200, 64
N = B * S
VOCAB = 100000
AGE_VOCAB = 120
TYPE_VOCAB = 2
EPS = 1e-12

SEG = 5
NSEG = N // SEG

NC, NS = 2, 16
NW = NC * NS
PER_W = NSEG // NW
PAIR = 1024
HALF = 512
NPAIR = PER_W // PAIR
KSUB = 4


def _gather_body(concept_hbm, cidx_hbm, out_hbm, idx_v, buf0, buf1,
                 sg0, sg1, sw0, sw1):
    wid = lax.axis_index("s") * NC + lax.axis_index("c")

    def pair_body(p, _):
        base = pl.multiple_of(wid * PER_W + p * PAIR, 8)
        krow = pl.multiple_of(base // 128, 8)
        pltpu.sync_copy(cidx_hbm.at[pl.ds(krow, 8)], idx_v)

        g0 = [pltpu.async_copy(concept_hbm.at[idx_v.at[k]],
                               buf0.at[pl.ds(k * 128, 128)], sg0)
              for k in range(KSUB)]
        g1 = [pltpu.async_copy(concept_hbm.at[idx_v.at[KSUB + k]],
                               buf1.at[pl.ds(k * 128, 128)], sg1)
              for k in range(KSUB)]
        for cp in g0:
            cp.wait()
        w0 = pltpu.async_copy(buf0, out_hbm.at[pl.ds(base, HALF)], sw0)
        for cp in g1:
            cp.wait()
        w1 = pltpu.async_copy(buf1, out_hbm.at[pl.ds(base + HALF, HALF)], sw1)
        w0.wait()
        w1.wait()
        return 0

    lax.fori_loop(0, NPAIR, pair_body, 0)


_sc_gather = functools.partial(
    pl.kernel,
    out_type=jax.ShapeDtypeStruct((NSEG, H), jnp.float32),
    mesh=plsc.VectorSubcoreMesh(core_axis_name="c", subcore_axis_name="s"),
    compiler_params=pltpu.CompilerParams(
        needs_layout_passes=False, use_tc_tiling_on_sc=False),
    scratch_types=[
        pltpu.VMEM((8, 128), jnp.int32),
        pltpu.VMEM((HALF, H), jnp.float32),
        pltpu.VMEM((HALF, H), jnp.float32),
        pltpu.SemaphoreType.DMA,
        pltpu.SemaphoreType.DMA,
        pltpu.SemaphoreType.DMA,
        pltpu.SemaphoreType.DMA,
    ],
)(_gather_body)


TB = 8192
NTB = NSEG // TB
IDR = TB // 128


def _ln_body(acc_ref, g_ref, pos_ref, tt_ref, age_ref, seg_ref, gam_ref,
             bet_ref, o_ref):
    del acc_ref
    pos3 = pos_ref[...][:, :, None]
    tt3 = tt_ref[...][:, :, None]

    iota3 = lax.broadcasted_iota(jnp.int32, (IDR, 128, AGE_VOCAB), 2)
    onehot = (pos3 == iota3).astype(jnp.bfloat16).reshape(TB, AGE_VOCAB)
    age_f = age_ref[...]
    age_hi = age_f.astype(jnp.bfloat16)
    age_lo = (age_f - age_hi.astype(jnp.float32)).astype(jnp.bfloat16)
    dn = (((1,), (0,)), ((), ()))
    age_part = (
        lax.dot_general(onehot, age_hi, dn,
                        preferred_element_type=jnp.float32)
        + lax.dot_general(onehot, age_lo, dn,
                          preferred_element_type=jnp.float32))

    s = seg_ref[...]
    seg_part = jnp.where(tt3 == 0, s[0:1, :][None], s[1:2, :][None])
    seg_part = seg_part.reshape(TB, H)

    x = g_ref[...] + age_part + seg_part
    mean = jnp.mean(x, axis=-1, keepdims=True)
    cx = x - mean
    var = jnp.mean(cx * cx, axis=-1, keepdims=True)
    y = cx * lax.rsqrt(var + EPS)
    o_ref[...] = y * gam_ref[...] + bet_ref[...]


def _make_tc_ln(seg_idx):
    return pl.pallas_call(
        _ln_body,
        grid=(NTB,),
        in_specs=[
            pl.BlockSpec(memory_space=pl.ANY),
            pl.BlockSpec((TB, H), lambda i: (i, 0)),
            pl.BlockSpec((IDR, 128), lambda i: (i, 0)),
            pl.BlockSpec((IDR, 128), lambda i: (i, 0)),
            pl.BlockSpec((AGE_VOCAB, H), lambda i: (0, 0)),
            pl.BlockSpec((TYPE_VOCAB, H), lambda i: (0, 0)),
            pl.BlockSpec((1, H), lambda i: (0, 0)),
            pl.BlockSpec((1, H), lambda i: (0, 0)),
        ],
        out_specs=pl.BlockSpec(
            (TB, H), lambda i, s=seg_idx: (s * NTB + i, 0)),
        out_shape=jax.ShapeDtypeStruct((N, H), jnp.float32),
        input_output_aliases={0: 0},
        compiler_params=pltpu.CompilerParams(
            dimension_semantics=("arbitrary",)),
    )


_tc_ln_calls = [_make_tc_ln(s5) for s5 in range(SEG)]


@jax.jit
def kernel(input_ids, token_type_ids, position_ids, concept_table,
           age_table, segment_table, ln_gamma, ln_beta):
    cidx = input_ids.astype(jnp.int32).reshape(N // 128, 128)
    pos = position_ids.astype(jnp.int32).reshape(N // 128, 128)
    tt = token_type_ids.astype(jnp.int32).reshape(N // 128, 128)
    gam = ln_gamma.reshape(1, H)
    bet = ln_beta.reshape(1, H)

    rows_per_seg = NSEG // 128
    gathered = [
        _sc_gather(concept_table,
                   lax.slice_in_dim(cidx, s5 * rows_per_seg,
                                    (s5 + 1) * rows_per_seg))
        for s5 in range(SEG)
    ]

    acc = jnp.zeros((N, H), jnp.float32)
    for s5 in range(SEG):
        sl = slice(s5 * rows_per_seg, (s5 + 1) * rows_per_seg)
        acc = _tc_ln_calls[s5](acc, gathered[s5], pos[sl], tt[sl],
                               age_table, segment_table, gam, bet)
    return acc.reshape(B, S, H)
